# Initial kernel scaffold; baseline (speedup 1.0000x reference)
import jax
import jax.numpy as jnp
from jax.experimental import pallas as pl


def kernel(l1_w, l2_w, l3_w, l4_w, l5_w, l6_w, fc_w, fc_b, x):
    raise NotImplementedError("write your pallas kernel here")


# speedup vs baseline: 1.0838x; 1.0838x over previous
import functools

import jax
import jax.numpy as jnp
from jax.experimental import pallas as pl
from jax.experimental.pallas import tpu as pltpu

_SLOPE = 0.01
_BINS = (1, 2, 4, 8, 16)
_VMEM = 64 * 1024 * 1024


def _lrelu(v):
    return jnp.where(v >= 0, v, _SLOPE * v)


def _conv3(x, w, *, pool):
    F, h, ww, cin = x.shape
    cout = w.shape[2]
    xb = jnp.pad(x.astype(jnp.bfloat16), ((0, 0), (1, 1), (1, 1), (0, 0)))
    acc = None
    for kh in range(3):
        rows = jnp.concatenate(
            [xb[:, kh:kh + h, kw:kw + ww, :] for kw in range(3)], axis=-1)
        part = jnp.dot(rows.reshape(F * h * ww, 3 * cin), w[kh],
                       preferred_element_type=jnp.float32)
        acc = part if acc is None else acc + part
    y = _lrelu(acc).reshape(F, h, ww, cout)
    if pool:
        y = y.reshape(F, h // 2, 2, ww // 2, 2, cout)
        y = jnp.maximum(jnp.maximum(y[:, :, 0, :, 0], y[:, :, 0, :, 1]),
                        jnp.maximum(y[:, :, 1, :, 0], y[:, :, 1, :, 1]))
    return y


def _stack_kernel(x_ref, w1_ref, w2_ref, w3_ref, w4_ref, w5_ref, w6_ref,
                  o_ref, *, fb, H, W):
    xb = x_ref[0, 0].astype(jnp.bfloat16)
    taps = [xb[:, i:i + H, j:j + W] for i in range(5) for j in range(5)]
    patch = jnp.stack(taps, axis=-1).reshape(fb * H * W, 25)
    y = jnp.dot(patch, w1_ref[...], preferred_element_type=jnp.float32)
    y = _lrelu(y).reshape(fb, H, W, 32)

    y = _conv3(y, w2_ref[...], pool=True)
    y = _conv3(y, w3_ref[...], pool=False)
    y = _conv3(y, w4_ref[...], pool=True)
    y = _conv3(y, w5_ref[...], pool=False)
    y = _conv3(y, w6_ref[...], pool=False)

    m = jnp.max(y, axis=0)
    j = pl.program_id(1)

    @pl.when(j == 0)
    def _():
        o_ref[0] = m

    @pl.when(j > 0)
    def _():
        o_ref[0] = jnp.maximum(o_ref[0], m)


def _conv_stack(x, ws, *, fb):
    n, s, H, W = x.shape
    Hp, Wp = H + 4, W + 4
    xp = jnp.pad(x, ((0, 0), (0, 0), (2, 2), (2, 2)))
    xp = xp.reshape(n, s // fb, fb, Hp, Wp)
    w_specs = [pl.BlockSpec(w.shape, lambda i, j, nd=w.ndim: (0,) * nd)
               for w in ws]
    kern = functools.partial(_stack_kernel, fb=fb, H=H, W=W)
    return pl.pallas_call(
        kern,
        out_shape=jax.ShapeDtypeStruct((n, H // 4, W // 4, 128), jnp.float32),
        grid=(n, s // fb),
        in_specs=[pl.BlockSpec((1, 1, fb, Hp, Wp), lambda i, j: (i, j, 0, 0, 0))]
        + w_specs,
        out_specs=pl.BlockSpec((1, H // 4, W // 4, 128),
                               lambda i, j: (i, 0, 0, 0)),
        compiler_params=pltpu.CompilerParams(
            dimension_semantics=("parallel", "arbitrary"),
            vmem_limit_bytes=_VMEM),
    )(xp, *ws)


def _head_kernel(g_ref, w_ref, b_ref, o_ref, acc_ref):
    k = pl.program_id(0)

    @pl.when(k == 0)
    def _():
        acc_ref[...] = jnp.zeros_like(acc_ref)

    acc_ref[...] += jnp.dot(g_ref[...].astype(jnp.bfloat16),
                            w_ref[...].astype(jnp.bfloat16),
                            preferred_element_type=jnp.float32)

    @pl.when(k == pl.num_programs(0) - 1)
    def _():
        z = acc_ref[...] + b_ref[...]
        dout = z.shape[1]
        off = 0
        for nb in _BINS:
            L = dout // nb
            ssum = z[:, :L]
            smax = z[:, :L]
            for b in range(1, nb):
                seg = z[:, b * L:(b + 1) * L]
                ssum = ssum + seg
                smax = jnp.maximum(smax, seg)
            o_ref[:, off:off + L] = ssum * (1.0 / nb) + smax
            off += L


def _head(g_flat, fc_w, fc_b):
    n, din = g_flat.shape
    dout = fc_w.shape[1]
    feat = sum(dout // b for b in _BINS)
    dk = max(d for d in (4096, 2048, 1024, 512, 256, 128, din)
             if din % d == 0 and d <= din)
    return pl.pallas_call(
        _head_kernel,
        out_shape=jax.ShapeDtypeStruct((n, feat), jnp.float32),
        grid=(din // dk,),
        in_specs=[pl.BlockSpec((n, dk), lambda k: (0, k)),
                  pl.BlockSpec((dk, dout), lambda k: (k, 0)),
                  pl.BlockSpec((1, dout), lambda k: (0, 0))],
        out_specs=pl.BlockSpec((n, feat), lambda k: (0, 0)),
        scratch_shapes=[pltpu.VMEM((n, dout), jnp.float32)],
        compiler_params=pltpu.CompilerParams(
            dimension_semantics=("arbitrary",),
            vmem_limit_bytes=_VMEM),
    )(g_flat, fc_w, fc_b.reshape(1, dout))


def kernel(l1_w, l2_w, l3_w, l4_w, l5_w, l6_w, fc_w, fc_b, x):
    n, s, H, W = x.shape
    fb = max(d for d in range(1, 7) if s % d == 0)
    bf = jnp.bfloat16
    ws = [
        l1_w.reshape(25, 32).astype(bf),
        l2_w.reshape(3, 3 * 32, 32).astype(bf),
        l3_w.reshape(3, 3 * 32, 64).astype(bf),
        l4_w.reshape(3, 3 * 64, 64).astype(bf),
        l5_w.reshape(3, 3 * 64, 128).astype(bf),
        l6_w.reshape(3, 3 * 128, 128).astype(bf),
    ]
    g = _conv_stack(x, ws, fb=fb)
    feat = _head(g.reshape(n, -1), fc_w, fc_b)
    return feat[:, None, :], None


# speedup vs baseline: 2.0522x; 1.8934x over previous
import functools

import jax
import jax.numpy as jnp
from jax.experimental import pallas as pl
from jax.experimental.pallas import tpu as pltpu

_SLOPE = 0.01
_BINS = (1, 2, 4, 8, 16)
_VMEM = 64 * 1024 * 1024


def _lrelu(v):
    return jnp.where(v >= 0, v, _SLOPE * v)


def _gconv(x, w, *, cin, pool):
    F, h, G, L = x.shape
    g = L // cin
    cout = w.shape[1] // g
    xb = x.astype(jnp.bfloat16)
    xp = jnp.pad(xb, ((0, 0), (1, 1), (1, 1), (0, 0)))
    pieces = []
    for kh in range(3):
        row = xp[:, kh:kh + h]
        pieces += [row[:, :, 0:G, L - cin:L],
                   row[:, :, 1:G + 1, :],
                   row[:, :, 2:G + 2, 0:cin]]
    patch = jnp.concatenate(pieces, axis=-1)
    y = jnp.dot(patch.reshape(F * h * G, patch.shape[-1]), w,
                preferred_element_type=jnp.float32)
    y = _lrelu(y).reshape(F, h, G, g * cout)
    if pool:
        y = y.reshape(F, h // 2, 2, G, g * cout)
        y = jnp.maximum(y[:, :, 0], y[:, :, 1])
        y = y.reshape(F, h // 2, G, g // 2, 2, cout)
        y = jnp.maximum(y[..., 0, :], y[..., 1, :])
        y = y.reshape(F, h // 2, G, (g // 2) * cout)
    return y


def _stack_kernel(x_ref, w1_ref, w2_ref, w3_ref, w4_ref, w5_ref, w6_ref,
                  o_ref, *, fb, H, W):
    x = x_ref[0, 0]
    G = W // 8
    xg = x.reshape(fb, H + 4, G, 8).astype(jnp.bfloat16)
    xp = jnp.pad(xg, ((0, 0), (0, 0), (1, 1), (0, 0)))
    pieces = []
    for kh in range(5):
        row = xp[:, kh:kh + H]
        pieces += [row[:, :, 0:G, 6:8],
                   row[:, :, 1:G + 1, :],
                   row[:, :, 2:G + 2, 0:2]]
    patch = jnp.concatenate(pieces, axis=-1)
    y = jnp.dot(patch.reshape(fb * H * G, 60), w1_ref[...],
                preferred_element_type=jnp.float32)
    y = _lrelu(y).reshape(fb, H, G, 8 * 32)

    y = _gconv(y, w2_ref[...], cin=32, pool=True)
    y = _gconv(y, w3_ref[...], cin=32, pool=False)
    y = _gconv(y, w4_ref[...], cin=64, pool=True)
    y = _gconv(y, w5_ref[...], cin=64, pool=False)
    y = _gconv(y, w6_ref[...], cin=128, pool=False)

    m = jnp.max(y, axis=0)
    j = pl.program_id(1)

    @pl.when(j == 0)
    def _():
        o_ref[0] = m

    @pl.when(j > 0)
    def _():
        o_ref[0] = jnp.maximum(o_ref[0], m)


def _conv_stack(x, ws, *, fb):
    n, s, H, W = x.shape
    G = W // 8
    xp = jnp.pad(x, ((0, 0), (0, 0), (2, 2), (0, 0)))
    xp = xp.reshape(n, s // fb, fb, H + 4, W)
    w_specs = [pl.BlockSpec(w.shape, lambda i, j, nd=w.ndim: (0,) * nd)
               for w in ws]
    kern = functools.partial(_stack_kernel, fb=fb, H=H, W=W)
    return pl.pallas_call(
        kern,
        out_shape=jax.ShapeDtypeStruct((n, H // 4, G, 256), jnp.float32),
        grid=(n, s // fb),
        in_specs=[pl.BlockSpec((1, 1, fb, H + 4, W),
                               lambda i, j: (i, j, 0, 0, 0))] + w_specs,
        out_specs=pl.BlockSpec((1, H // 4, G, 256), lambda i, j: (i, 0, 0, 0)),
        compiler_params=pltpu.CompilerParams(
            dimension_semantics=("parallel", "arbitrary"),
            vmem_limit_bytes=_VMEM),
    )(xp, *ws)


def _head_kernel(g_ref, w_ref, b_ref, o_ref, acc_ref):
    k = pl.program_id(0)

    @pl.when(k == 0)
    def _():
        acc_ref[...] = jnp.zeros_like(acc_ref)

    acc_ref[...] += jnp.dot(g_ref[...].astype(jnp.bfloat16),
                            w_ref[...].astype(jnp.bfloat16),
                            preferred_element_type=jnp.float32)

    @pl.when(k == pl.num_programs(0) - 1)
    def _():
        z = acc_ref[...] + b_ref[...]
        dout = z.shape[1]
        off = 0
        for nb in _BINS:
            L = dout // nb
            ssum = z[:, :L]
            smax = z[:, :L]
            for b in range(1, nb):
                seg = z[:, b * L:(b + 1) * L]
                ssum = ssum + seg
                smax = jnp.maximum(smax, seg)
            o_ref[:, off:off + L] = ssum * (1.0 / nb) + smax
            off += L


def _head(g_flat, fc_w, fc_b):
    n, din = g_flat.shape
    dout = fc_w.shape[1]
    feat = sum(dout // b for b in _BINS)
    dk = max(d for d in (4096, 2048, 1024, 512, 256, 128, din)
             if din % d == 0 and d <= din)
    return pl.pallas_call(
        _head_kernel,
        out_shape=jax.ShapeDtypeStruct((n, feat), jnp.float32),
        grid=(din // dk,),
        in_specs=[pl.BlockSpec((n, dk), lambda k: (0, k)),
                  pl.BlockSpec((dk, dout), lambda k: (k, 0)),
                  pl.BlockSpec((1, dout), lambda k: (0, 0))],
        out_specs=pl.BlockSpec((n, feat), lambda k: (0, 0)),
        scratch_shapes=[pltpu.VMEM((n, dout), jnp.float32)],
        compiler_params=pltpu.CompilerParams(
            dimension_semantics=("arbitrary",),
            vmem_limit_bytes=_VMEM),
    )(g_flat, fc_w, fc_b.reshape(1, dout))


def _gw3(w, g):
    _, _, cin, cout = w.shape
    out = jnp.zeros((3, g + 2, cin, g, cout), w.dtype)
    for p in range(g):
        for k in range(3):
            out = out.at[:, p + k, :, p, :].set(w[:, k, :, :])
    return out.reshape(3 * (g + 2) * cin, g * cout).astype(jnp.bfloat16)


def _gw1(w, g):
    _, _, _, cout = w.shape
    out = jnp.zeros((5, g + 4, g, cout), w.dtype)
    for p in range(g):
        for k in range(5):
            out = out.at[:, p + k, p, :].set(w[:, k, 0, :])
    return out.reshape(5 * (g + 4), g * cout).astype(jnp.bfloat16)


def kernel(l1_w, l2_w, l3_w, l4_w, l5_w, l6_w, fc_w, fc_b, x):
    n, s, H, W = x.shape
    fb = max(d for d in range(1, 7) if s % d == 0)
    ws = [
        _gw1(l1_w, 8),
        _gw3(l2_w, 8),
        _gw3(l3_w, 4),
        _gw3(l4_w, 4),
        _gw3(l5_w, 2),
        _gw3(l6_w, 2),
    ]
    g = _conv_stack(x, ws, fb=fb)
    feat = _head(g.reshape(n, -1), fc_w, fc_b)
    return feat[:, None, :], None


# speedup vs baseline: 2.6760x; 1.3040x over previous
import functools

import jax
import jax.numpy as jnp
from jax.experimental import pallas as pl
from jax.experimental.pallas import tpu as pltpu

_SLOPE = 0.01
_BINS = (1, 2, 4, 8, 16)
_VMEM = 64 * 1024 * 1024


def _lrelu(v):
    return jnp.where(v >= 0, v, _SLOPE * v)


def _gconv(x, w, *, cin, nout, left_slot, pool):
    F, h, G, L = x.shape
    cout = w.shape[1] // nout
    xb = x.astype(jnp.bfloat16)
    xp = jnp.pad(xb, ((0, 0), (1, 1), (1, 1), (0, 0)))
    pieces = []
    for kh in range(3):
        row = xp[:, kh:kh + h]
        pieces += [row[:, :, 0:G, left_slot * cin:(left_slot + 1) * cin],
                   row[:, :, 1:G + 1, :],
                   row[:, :, 2:G + 2, 0:cin]]
    patch = jnp.concatenate(pieces, axis=-1)
    y = jnp.dot(patch.reshape(F * h * G, patch.shape[-1]), w,
                preferred_element_type=jnp.float32)
    y = _lrelu(y).reshape(F, h, G, nout * cout)
    if pool:
        y = y.reshape(F, h // 2, 2, G, nout * cout)
        y = jnp.maximum(y[:, :, 0], y[:, :, 1])
        y = jnp.maximum(y, jnp.roll(y, -cout, axis=-1))
    return y


def _stack_kernel(x_ref, w1_ref, w2_ref, w3_ref, w4_ref, w5_ref, w6_ref,
                  o_ref, *, fb, H, W):
    x = x_ref[0, 0]
    G = W // 8
    xg = x.reshape(fb, H + 4, G, 8).astype(jnp.bfloat16)
    xp = jnp.pad(xg, ((0, 0), (0, 0), (1, 1), (0, 0)))
    pieces = []
    for kh in range(5):
        row = xp[:, kh:kh + H]
        pieces += [row[:, :, 0:G, 6:8],
                   row[:, :, 1:G + 1, :],
                   row[:, :, 2:G + 2, 0:2]]
    patch = jnp.concatenate(pieces, axis=-1)
    y = jnp.dot(patch.reshape(fb * H * G, 60), w1_ref[...],
                preferred_element_type=jnp.float32)
    y = _lrelu(y).reshape(fb, H, G, 8 * 32)

    y = _gconv(y, w2_ref[...], cin=32, nout=8, left_slot=7, pool=True)
    y = _gconv(y, w3_ref[...], cin=32, nout=4, left_slot=6, pool=False)
    y = _gconv(y, w4_ref[...], cin=64, nout=4, left_slot=3, pool=True)
    y = _gconv(y, w5_ref[...], cin=64, nout=2, left_slot=2, pool=False)
    y = _gconv(y, w6_ref[...], cin=128, nout=2, left_slot=1, pool=False)

    m = jnp.max(y, axis=0)
    j = pl.program_id(1)

    @pl.when(j == 0)
    def _():
        o_ref[0] = m

    @pl.when(j > 0)
    def _():
        o_ref[0] = jnp.maximum(o_ref[0], m)


def _conv_stack(x, ws, *, fb):
    n, s, H, W = x.shape
    G = W // 8
    xp = jnp.pad(x, ((0, 0), (0, 0), (2, 2), (0, 0)))
    xp = xp.reshape(n, s // fb, fb, H + 4, W)
    w_specs = [pl.BlockSpec(w.shape, lambda i, j, nd=w.ndim: (0,) * nd)
               for w in ws]
    kern = functools.partial(_stack_kernel, fb=fb, H=H, W=W)
    return pl.pallas_call(
        kern,
        out_shape=jax.ShapeDtypeStruct((n, H // 4, G, 256), jnp.float32),
        grid=(n, s // fb),
        in_specs=[pl.BlockSpec((1, 1, fb, H + 4, W),
                               lambda i, j: (i, j, 0, 0, 0))] + w_specs,
        out_specs=pl.BlockSpec((1, H // 4, G, 256), lambda i, j: (i, 0, 0, 0)),
        compiler_params=pltpu.CompilerParams(
            dimension_semantics=("parallel", "arbitrary"),
            vmem_limit_bytes=_VMEM),
    )(xp, *ws)


def _head_kernel(g_ref, w_ref, b_ref, o_ref, acc_ref):
    k = pl.program_id(0)

    @pl.when(k == 0)
    def _():
        acc_ref[...] = jnp.zeros_like(acc_ref)

    acc_ref[...] += jnp.dot(g_ref[...].astype(jnp.bfloat16),
                            w_ref[...].astype(jnp.bfloat16),
                            preferred_element_type=jnp.float32)

    @pl.when(k == pl.num_programs(0) - 1)
    def _():
        z = acc_ref[...] + b_ref[...]
        dout = z.shape[1]
        off = 0
        for nb in _BINS:
            L = dout // nb
            ssum = z[:, :L]
            smax = z[:, :L]
            for b in range(1, nb):
                seg = z[:, b * L:(b + 1) * L]
                ssum = ssum + seg
                smax = jnp.maximum(smax, seg)
            o_ref[:, off:off + L] = ssum * (1.0 / nb) + smax
            off += L


def _head(g_flat, fc_w, fc_b):
    n, din = g_flat.shape
    dout = fc_w.shape[1]
    feat = sum(dout // b for b in _BINS)
    dk = max(d for d in (4096, 2048, 1024, 512, 256, 128, din)
             if din % d == 0 and d <= din)
    return pl.pallas_call(
        _head_kernel,
        out_shape=jax.ShapeDtypeStruct((n, feat), jnp.float32),
        grid=(din // dk,),
        in_specs=[pl.BlockSpec((n, dk), lambda k: (0, k)),
                  pl.BlockSpec((dk, dout), lambda k: (k, 0)),
                  pl.BlockSpec((1, dout), lambda k: (0, 0))],
        out_specs=pl.BlockSpec((n, feat), lambda k: (0, 0)),
        scratch_shapes=[pltpu.VMEM((n, dout), jnp.float32)],
        compiler_params=pltpu.CompilerParams(
            dimension_semantics=("arbitrary",),
            vmem_limit_bytes=_VMEM),
    )(g_flat, fc_w, fc_b.reshape(1, dout))


def _gw(w, gout, S, sigma):
    _, _, cin, cout = w.shape
    out = jnp.zeros((3, S + 2, cin, gout, cout), w.dtype)
    for j in range(gout):
        for k in range(3):
            d = sigma * (j - 1 + k)
            ds = 0 if d < 0 else (S + 1 if d >= S else 1 + d)
            out = out.at[:, ds, :, j, :].set(w[:, k, :, :])
    return out.reshape(3 * (S + 2) * cin, gout * cout).astype(jnp.bfloat16)


def _gw1(w, g):
    _, _, _, cout = w.shape
    out = jnp.zeros((5, g + 4, g, cout), w.dtype)
    for p in range(g):
        for k in range(5):
            out = out.at[:, p + k, p, :].set(w[:, k, 0, :])
    return out.reshape(5 * (g + 4), g * cout).astype(jnp.bfloat16)


def kernel(l1_w, l2_w, l3_w, l4_w, l5_w, l6_w, fc_w, fc_b, x):
    n, s, H, W = x.shape
    fb = max(d for d in range(1, 7) if s % d == 0)
    ws = [
        _gw1(l1_w, 8),
        _gw(l2_w, 8, 8, 1),
        _gw(l3_w, 4, 8, 2),
        _gw(l4_w, 4, 4, 1),
        _gw(l5_w, 2, 4, 2),
        _gw(l6_w, 2, 2, 1),
    ]
    g = _conv_stack(x, ws, fb=fb)
    feat = _head(g.reshape(n, -1), fc_w, fc_b)
    return feat[:, None, :], None
